# baseline (speedup 1.0000x reference)
import functools

import jax
import jax.numpy as jnp
from jax import lax
from jax.experimental import pallas as pl
from jax.experimental.pallas import tpu as pltpu
from jax.experimental.pallas import tpu_sc as plsc

N = 10000
E = 320000
D = 128
D_E = 16

NC = 2
NS = 16
NW = NC * NS

CHUNK = 128
ACC_ROWS = 10240
ROWS_PER_TILE = ACC_ROWS // NS

MM_BLK = 1000


def _mm_body(h_ref, aw_ref, ab_ref, bw_ref, bb_ref, ah_ref, bh_ref):
    hb = h_ref[...]
    ah_ref[...] = jnp.dot(hb, aw_ref[...],
                          preferred_element_type=jnp.float32) + ab_ref[...]
    bh_ref[...] = jnp.dot(hb, bw_ref[...],
                          preferred_element_type=jnp.float32) + bb_ref[...]


def _matmuls(h, A_W, A_b2, B_W, B_b2):
    grid = (N // MM_BLK,)
    full = pl.BlockSpec((D, D), lambda i: (0, 0))
    vec = pl.BlockSpec((1, D), lambda i: (0, 0))
    blk = pl.BlockSpec((MM_BLK, D), lambda i: (i, 0))
    return pl.pallas_call(
        _mm_body,
        grid=grid,
        in_specs=[blk, full, vec, full, vec],
        out_specs=[blk, blk],
        out_shape=[jax.ShapeDtypeStruct((N, D), jnp.float32)] * 2,
    )(h, A_W, A_b2, B_W, B_b2)


NB = 2
NID = 4
NCHUNKS = E // CHUNK
WCH = 80
LASTW = NCHUNKS - (NW - 1) * WCH


def _sc_body(bh, ei, out, sidx, didx, rows, acc, gsem, ssem, isem):
    c = lax.axis_index("c")
    s = lax.axis_index("s")
    wid = c * NS + s

    def _zb(i, _):
        rows[0, i // 8, pl.ds((i % 8) * 16, 16)] = jnp.zeros((16,),
                                                             jnp.float32)
        return _
    lax.fori_loop(0, CHUNK * 8, _zb, None)

    def _z(j, _):
        pltpu.sync_copy(rows.at[0],
                        acc.at[pl.ds(s * ROWS_PER_TILE + j * CHUNK, CHUNK)])
        return _
    lax.fori_loop(0, ROWS_PER_TILE // CHUNK, _z, None)

    plsc.subcore_barrier()

    base = wid * WCH
    ncw = jnp.where(wid == NW - 1, LASTW, WCH)

    def _issue_idx(kk, sl):
        off = (base + kk) * CHUNK
        pltpu.async_copy(ei.at[0, pl.ds(off, CHUNK)], sidx.at[sl],
                         isem.at[sl])
        pltpu.async_copy(ei.at[1, pl.ds(off, CHUNK)], didx.at[sl],
                         isem.at[sl])

    def _wait_idx(kk, sl):
        off = (base + kk) * CHUNK
        pltpu.make_async_copy(ei.at[0, pl.ds(off, CHUNK)], sidx.at[sl],
                              isem.at[sl]).wait()
        pltpu.make_async_copy(ei.at[1, pl.ds(off, CHUNK)], didx.at[sl],
                              isem.at[sl]).wait()

    for t in range(3):
        _issue_idx(t, t)
    _wait_idx(0, 0)
    pltpu.async_copy(bh.at[sidx.at[0]], rows.at[0], gsem.at[0])

    def _edge(j, _):
        for b in range(NID):
            kk = j * NID + b
            r = b % NB
            rn = (b + 1) % NB
            sl_n = (b + 1) % NID
            sl_i = (b + 3) % NID
            pltpu.make_async_copy(bh.at[sidx.at[b]], rows.at[r],
                                  gsem.at[r]).wait()
            pltpu.async_copy(rows.at[r], acc.at[didx.at[b]], ssem.at[r],
                             add=True)

            @pl.when((kk >= 1) & (kk + 1 < ncw))
            def _():
                pltpu.make_async_copy(rows.at[rn], acc.at[didx.at[b]],
                                      ssem.at[rn]).wait()

            @pl.when(kk + 3 < ncw)
            def _():
                _issue_idx(kk + 3, sl_i)

            @pl.when(kk + 1 < ncw)
            def _():
                _wait_idx(kk + 1, sl_n)
                pltpu.async_copy(bh.at[sidx.at[sl_n]], rows.at[rn],
                                 gsem.at[rn])
        return _
    lax.fori_loop(0, ncw // NID, _edge, None)

    for t in range(NB):
        pltpu.make_async_copy(rows.at[t], acc.at[didx.at[t]],
                              ssem.at[t]).wait()

    plsc.subcore_barrier()

    def _cp(j, _):
        r0 = s * ROWS_PER_TILE + j * CHUNK
        pltpu.sync_copy(acc.at[pl.ds(r0, CHUNK)], rows.at[0])
        pltpu.sync_copy(rows.at[0], out.at[c, pl.ds(r0, CHUNK)])
        return _
    lax.fori_loop(0, ROWS_PER_TILE // CHUNK, _cp, None)


def _sc_aggregate(bh, edge_index):
    mesh = plsc.VectorSubcoreMesh(core_axis_name="c", subcore_axis_name="s")
    fn = pl.kernel(
        _sc_body,
        mesh=mesh,
        scratch_types=[
            pltpu.VMEM((NID, CHUNK), jnp.int32),
            pltpu.VMEM((NID, CHUNK), jnp.int32),
            pltpu.VMEM((NB, CHUNK, D), jnp.float32),
            pltpu.VMEM_SHARED((ACC_ROWS, D), jnp.float32),
            pltpu.SemaphoreType.DMA((NB,)),
            pltpu.SemaphoreType.DMA((NB,)),
            pltpu.SemaphoreType.DMA((NID,)),
        ],
        out_type=jax.ShapeDtypeStruct((NC, ACC_ROWS, D), jnp.float32),
    )
    return fn(bh, edge_index)


def _ecopy_body(e_ref, out_ref):
    out_ref[...] = e_ref[...]


def _ecopy(e):
    blk = pl.BlockSpec((E // 160, D_E), lambda i: (i, 0))
    return pl.pallas_call(
        _ecopy_body,
        grid=(160,),
        in_specs=[blk],
        out_specs=blk,
        out_shape=jax.ShapeDtypeStruct((E, D_E), jnp.float32),
    )(e)


def _final_body(h_ref, ah_ref, p_ref, g_ref, b_ref, out_ref):
    x = ah_ref[...] + p_ref[0, :N] + p_ref[1, :N]
    mean = jnp.mean(x, axis=0, keepdims=True)
    var = jnp.mean(x * x, axis=0, keepdims=True) - mean * mean
    xn = (x - mean) * (lax.rsqrt(var + 1e-5) * g_ref[...]) + b_ref[...]
    out_ref[...] = h_ref[...] + jnp.maximum(xn, 0.0)


def _finalize(h, ah, parts, gamma2, beta2):
    full = pl.BlockSpec((N, D), lambda: (0, 0))
    pblk = pl.BlockSpec((NC, ACC_ROWS, D), lambda: (0, 0, 0))
    vec = pl.BlockSpec((1, D), lambda: (0, 0))
    return pl.pallas_call(
        _final_body,
        in_specs=[full, full, pblk, vec, vec],
        out_specs=full,
        out_shape=jax.ShapeDtypeStruct((N, D), jnp.float32),
    )(h, ah, parts, gamma2, beta2)


@functools.partial(jax.jit)
def kernel(h, edge_index, e, A_W, A_b, B_W, B_b, gamma, beta):
    ah, bh = _matmuls(h, A_W, A_b.reshape(1, D), B_W, B_b.reshape(1, D))
    parts = _sc_aggregate(bh, edge_index)
    e_out = _ecopy(e)
    out = _finalize(h, ah, parts, gamma.reshape(1, D), beta.reshape(1, D))
    return (out, e_out)


# speedup vs baseline: 2.0764x; 2.0764x over previous
import functools

import jax
import jax.numpy as jnp
from jax import lax
from jax.experimental import pallas as pl
from jax.experimental.pallas import tpu as pltpu
from jax.experimental.pallas import tpu_sc as plsc

N = 10000
E = 320000
D = 128
D_E = 16

NC = 2
NS = 16
NW = NC * NS

CHUNK = 128
ACC_ROWS = 10240
ROWS_PER_TILE = ACC_ROWS // NS

MM_BLK = 1000


def _mm_body(h_ref, aw_ref, ab_ref, bw_ref, bb_ref, ah_ref, bh_ref):
    hb = h_ref[...]
    ah_ref[...] = jnp.dot(hb, aw_ref[...],
                          preferred_element_type=jnp.float32) + ab_ref[...]
    bh_ref[...] = jnp.dot(hb, bw_ref[...],
                          preferred_element_type=jnp.float32) + bb_ref[...]


def _matmuls(h, A_W, A_b2, B_W, B_b2):
    grid = (N // MM_BLK,)
    full = pl.BlockSpec((D, D), lambda i: (0, 0))
    vec = pl.BlockSpec((1, D), lambda i: (0, 0))
    blk = pl.BlockSpec((MM_BLK, D), lambda i: (i, 0))
    return pl.pallas_call(
        _mm_body,
        grid=grid,
        in_specs=[blk, full, vec, full, vec],
        out_specs=[blk, blk],
        out_shape=[jax.ShapeDtypeStruct((N, D), jnp.float32)] * 2,
    )(h, A_W, A_b2, B_W, B_b2)


NB = 2
NCHUNKS = E // CHUNK
BCH = 40
NBLK = (NCHUNKS + BCH - 1) // BCH
LAST_BCH = NCHUNKS - (NBLK - 1) * BCH


def _sc_body(bh, ei, out, sidx, didx, rows, acc, gsem, ssem):
    c = lax.axis_index("c")
    s = lax.axis_index("s")
    wid = c * NS + s

    def _zb(i, _):
        rows[0, i // 8, pl.ds((i % 8) * 16, 16)] = jnp.zeros((16,),
                                                             jnp.float32)
        return _
    lax.fori_loop(0, CHUNK * 8, _zb, None)

    def _z(j, _):
        pltpu.sync_copy(rows.at[0],
                        acc.at[pl.ds(s * ROWS_PER_TILE + j * CHUNK, CHUNK)])
        return _
    lax.fori_loop(0, ROWS_PER_TILE // CHUNK, _z, None)

    plsc.subcore_barrier()

    def _sl(ref, kk):
        return ref.at[pl.ds(kk * CHUNK, CHUNK)]

    for t2 in range(2):
        t = wid + t2 * NW

        @pl.when(t < NBLK)
        def _():
            nh = jnp.where(t == NBLK - 1, LAST_BCH, BCH)
            off = t * BCH * CHUNK

            @pl.when(t == NBLK - 1)
            def _():
                pltpu.sync_copy(ei.at[0, pl.ds(off, LAST_BCH * CHUNK)],
                                sidx.at[pl.ds(0, LAST_BCH * CHUNK)])
                pltpu.sync_copy(ei.at[1, pl.ds(off, LAST_BCH * CHUNK)],
                                didx.at[pl.ds(0, LAST_BCH * CHUNK)])

            @pl.when(t != NBLK - 1)
            def _():
                pltpu.sync_copy(ei.at[0, pl.ds(off, BCH * CHUNK)], sidx)
                pltpu.sync_copy(ei.at[1, pl.ds(off, BCH * CHUNK)], didx)

            for b in range(NB):
                pltpu.async_copy(bh.at[_sl(sidx, b)], rows.at[b],
                                 gsem.at[b])

            def _edge(j, _):
                for b in range(NB):
                    kk = j * NB + b
                    bp = (b + 1) % NB
                    pltpu.make_async_copy(bh.at[_sl(sidx, kk)], rows.at[b],
                                          gsem.at[b]).wait()
                    pltpu.async_copy(rows.at[b], acc.at[_sl(didx, kk)],
                                     ssem.at[b], add=True)

                    @pl.when((kk >= 1) & (kk + 1 < nh))
                    def _():
                        pltpu.make_async_copy(rows.at[bp],
                                              acc.at[_sl(didx, kk)],
                                              ssem.at[bp]).wait()
                        pltpu.async_copy(bh.at[_sl(sidx, kk + 1)],
                                         rows.at[bp], gsem.at[bp])
                return _
            lax.fori_loop(0, nh // NB, _edge, None)

            for b in range(NB):
                pltpu.make_async_copy(rows.at[b], acc.at[_sl(didx, b)],
                                      ssem.at[b]).wait()

    plsc.subcore_barrier()

    def _cp(j, _):
        r0 = s * ROWS_PER_TILE + j * CHUNK
        pltpu.sync_copy(acc.at[pl.ds(r0, CHUNK)], rows.at[0])
        pltpu.sync_copy(rows.at[0], out.at[c, pl.ds(r0, CHUNK)])
        return _
    lax.fori_loop(0, ROWS_PER_TILE // CHUNK, _cp, None)


def _sc_aggregate(bh, edge_index):
    mesh = plsc.VectorSubcoreMesh(core_axis_name="c", subcore_axis_name="s")
    fn = pl.kernel(
        _sc_body,
        mesh=mesh,
        scratch_types=[
            pltpu.VMEM((BCH * CHUNK,), jnp.int32),
            pltpu.VMEM((BCH * CHUNK,), jnp.int32),
            pltpu.VMEM((NB, CHUNK, D), jnp.float32),
            pltpu.VMEM_SHARED((ACC_ROWS, D), jnp.float32),
            pltpu.SemaphoreType.DMA((NB,)),
            pltpu.SemaphoreType.DMA((NB,)),
        ],
        out_type=jax.ShapeDtypeStruct((NC, ACC_ROWS, D), jnp.float32),
    )
    return fn(bh, edge_index)


def _final_body(h_ref, ah_ref, p_ref, g_ref, b_ref, out_ref):
    x = ah_ref[...] + p_ref[0, :N] + p_ref[1, :N]
    mean = jnp.mean(x, axis=0, keepdims=True)
    var = jnp.mean(x * x, axis=0, keepdims=True) - mean * mean
    xn = (x - mean) * (lax.rsqrt(var + 1e-5) * g_ref[...]) + b_ref[...]
    out_ref[...] = h_ref[...] + jnp.maximum(xn, 0.0)


def _finalize(h, ah, parts, gamma2, beta2):
    full = pl.BlockSpec((N, D), lambda: (0, 0))
    pblk = pl.BlockSpec((NC, ACC_ROWS, D), lambda: (0, 0, 0))
    vec = pl.BlockSpec((1, D), lambda: (0, 0))
    return pl.pallas_call(
        _final_body,
        in_specs=[full, full, pblk, vec, vec],
        out_specs=full,
        out_shape=jax.ShapeDtypeStruct((N, D), jnp.float32),
    )(h, ah, parts, gamma2, beta2)


@functools.partial(jax.jit)
def kernel(h, edge_index, e, A_W, A_b, B_W, B_b, gamma, beta):
    ah, bh = _matmuls(h, A_W, A_b.reshape(1, D), B_W, B_b.reshape(1, D))
    parts = _sc_aggregate(bh, edge_index)
    out = _finalize(h, ah, parts, gamma.reshape(1, D), beta.reshape(1, D))
    return (out, e)


# speedup vs baseline: 2.1191x; 1.0206x over previous
import functools

import jax
import jax.numpy as jnp
from jax import lax
from jax.experimental import pallas as pl
from jax.experimental.pallas import tpu as pltpu
from jax.experimental.pallas import tpu_sc as plsc

N = 10000
E = 320000
D = 128
D_E = 16

NC = 2
NS = 16
NW = NC * NS

CHUNK = 128
ACC_ROWS = 10240
ROWS_PER_TILE = ACC_ROWS // NS

MM_BLK = 1000


def _mm_body(h_ref, aw_ref, ab_ref, bw_ref, bb_ref, ah_ref, bh_ref):
    hb = h_ref[...]
    ah_ref[...] = jnp.dot(hb, aw_ref[...],
                          preferred_element_type=jnp.float32) + ab_ref[...]
    bh_ref[...] = jnp.dot(hb, bw_ref[...],
                          preferred_element_type=jnp.float32) + bb_ref[...]


def _matmuls(h, A_W, A_b2, B_W, B_b2):
    grid = (N // MM_BLK,)
    full = pl.BlockSpec((D, D), lambda i: (0, 0))
    vec = pl.BlockSpec((1, D), lambda i: (0, 0))
    blk = pl.BlockSpec((MM_BLK, D), lambda i: (i, 0))
    return pl.pallas_call(
        _mm_body,
        grid=grid,
        in_specs=[blk, full, vec, full, vec],
        out_specs=[blk, blk],
        out_shape=[jax.ShapeDtypeStruct((N, D), jnp.float32)] * 2,
    )(h, A_W, A_b2, B_W, B_b2)


NB = 2
NID = 4
NCHUNKS = E // CHUNK
WCH = 80
LASTW = NCHUNKS - (NW - 1) * WCH


def _sc_body(bh, ei, out, sidx, didx, rows, acc, gsem, ssem, isem):
    c = lax.axis_index("c")
    s = lax.axis_index("s")
    wid = c * NS + s

    def _zb(i, _):
        rows[0, i // 8, pl.ds((i % 8) * 16, 16)] = jnp.zeros((16,),
                                                             jnp.float32)
        return _
    lax.fori_loop(0, CHUNK * 8, _zb, None)

    def _z(j, _):
        pltpu.sync_copy(rows.at[0],
                        acc.at[pl.ds(s * ROWS_PER_TILE + j * CHUNK, CHUNK)])
        return _
    lax.fori_loop(0, ROWS_PER_TILE // CHUNK, _z, None)

    plsc.subcore_barrier()

    base = wid * WCH
    ncw = jnp.where(wid == NW - 1, LASTW, WCH)

    def _issue_idx(kk, sl):
        off = (base + kk) * CHUNK
        pltpu.async_copy(ei.at[0, pl.ds(off, CHUNK)], sidx.at[sl],
                         isem.at[sl])
        pltpu.async_copy(ei.at[1, pl.ds(off, CHUNK)], didx.at[sl],
                         isem.at[sl])

    def _wait_idx(kk, sl):
        off = (base + kk) * CHUNK
        pltpu.make_async_copy(ei.at[0, pl.ds(off, CHUNK)], sidx.at[sl],
                              isem.at[sl]).wait()
        pltpu.make_async_copy(ei.at[1, pl.ds(off, CHUNK)], didx.at[sl],
                              isem.at[sl]).wait()

    for t in range(3):
        _issue_idx(t, t)
    _wait_idx(0, 0)
    pltpu.async_copy(bh.at[sidx.at[0]], rows.at[0], gsem.at[0])

    def _edge(j, _):
        for b in range(NID):
            kk = j * NID + b
            r = b % NB
            rn = (b + 1) % NB
            sl_n = (b + 1) % NID
            sl_i = (b + 3) % NID
            pltpu.make_async_copy(bh.at[sidx.at[b]], rows.at[r],
                                  gsem.at[r]).wait()
            pltpu.async_copy(rows.at[r], acc.at[didx.at[b]], ssem.at[r],
                             add=True)

            @pl.when((kk >= 1) & (kk + 1 < ncw))
            def _():
                pltpu.make_async_copy(rows.at[rn], acc.at[didx.at[b]],
                                      ssem.at[rn]).wait()

            @pl.when(kk + 3 < ncw)
            def _():
                _issue_idx(kk + 3, sl_i)

            @pl.when(kk + 1 < ncw)
            def _():
                _wait_idx(kk + 1, sl_n)
                pltpu.async_copy(bh.at[sidx.at[sl_n]], rows.at[rn],
                                 gsem.at[rn])
        return _
    lax.fori_loop(0, ncw // NID, _edge, None)

    for t in range(NB):
        pltpu.make_async_copy(rows.at[t], acc.at[didx.at[t]],
                              ssem.at[t]).wait()

    plsc.subcore_barrier()

    def _cp(j, _):
        r0 = s * ROWS_PER_TILE + j * CHUNK
        pltpu.sync_copy(acc.at[pl.ds(r0, CHUNK)], rows.at[0])
        pltpu.sync_copy(rows.at[0], out.at[c, pl.ds(r0, CHUNK)])
        return _
    lax.fori_loop(0, ROWS_PER_TILE // CHUNK, _cp, None)


def _sc_aggregate(bh, edge_index):
    mesh = plsc.VectorSubcoreMesh(core_axis_name="c", subcore_axis_name="s")
    fn = pl.kernel(
        _sc_body,
        mesh=mesh,
        scratch_types=[
            pltpu.VMEM((NID, CHUNK), jnp.int32),
            pltpu.VMEM((NID, CHUNK), jnp.int32),
            pltpu.VMEM((NB, CHUNK, D), jnp.float32),
            pltpu.VMEM_SHARED((ACC_ROWS, D), jnp.float32),
            pltpu.SemaphoreType.DMA((NB,)),
            pltpu.SemaphoreType.DMA((NB,)),
            pltpu.SemaphoreType.DMA((NID,)),
        ],
        out_type=jax.ShapeDtypeStruct((NC, ACC_ROWS, D), jnp.float32),
    )
    return fn(bh, edge_index)


def _final_body(h_ref, ah_ref, p_ref, g_ref, b_ref, out_ref):
    x = ah_ref[...] + p_ref[0, :N] + p_ref[1, :N]
    mean = jnp.mean(x, axis=0, keepdims=True)
    var = jnp.mean(x * x, axis=0, keepdims=True) - mean * mean
    xn = (x - mean) * (lax.rsqrt(var + 1e-5) * g_ref[...]) + b_ref[...]
    out_ref[...] = h_ref[...] + jnp.maximum(xn, 0.0)


def _finalize(h, ah, parts, gamma2, beta2):
    full = pl.BlockSpec((N, D), lambda: (0, 0))
    pblk = pl.BlockSpec((NC, ACC_ROWS, D), lambda: (0, 0, 0))
    vec = pl.BlockSpec((1, D), lambda: (0, 0))
    return pl.pallas_call(
        _final_body,
        in_specs=[full, full, pblk, vec, vec],
        out_specs=full,
        out_shape=jax.ShapeDtypeStruct((N, D), jnp.float32),
    )(h, ah, parts, gamma2, beta2)


@functools.partial(jax.jit)
def kernel(h, edge_index, e, A_W, A_b, B_W, B_b, gamma, beta):
    ah, bh = _matmuls(h, A_W, A_b.reshape(1, D), B_W, B_b.reshape(1, D))
    parts = _sc_aggregate(bh, edge_index)
    out = _finalize(h, ah, parts, gamma.reshape(1, D), beta.reshape(1, D))
    return (out, e)
